# Initial kernel scaffold; baseline (speedup 1.0000x reference)
import jax
import jax.numpy as jnp
from jax.experimental import pallas as pl


def kernel(features, indices, params):
    raise NotImplementedError("write your pallas kernel here")


# speedup vs baseline: 1.9785x; 1.9785x over previous
import jax
import jax.numpy as jnp
from jax.experimental import pallas as pl
from jax.experimental.pallas import tpu as pltpu

_INTERPRET = False
_B, _H, _W, _CIN, _BASE = 4, 125, 125, 8, 32
_HP = 128
_F32 = jnp.float32
_BF16 = jnp.bfloat16


def _bd(w):
    return jax.scipy.linalg.block_diag(w, w, w, w)


def _taps_bd(w):
    return [_bd(w[ky, kx]) for ky in range(3) for kx in range(3)]


def _taps(w):
    return [w[ky, kx] for ky in range(3) for kx in range(3)]


def _t4(v):
    return jnp.tile(v.reshape(1, -1), (1, 4))


def _wrap_slice(v, start, size):
    n, c = v.shape
    start %= n
    if start + size <= n:
        return jax.lax.slice(v, (start, 0), (start + size, c))
    k = n - start
    return jnp.concatenate([jax.lax.slice(v, (start, 0), (n, c)),
                            jax.lax.slice(v, (0, 0), (size - k, c))], axis=0)


def _roll_conv(x2, taps, wb):
    n = x2.shape[0]
    cout = taps[0].shape[1]
    nchunk = max(1, n // 4096)
    cs = n // nchunk
    outs = []
    for ci in range(nchunk):
        r0 = ci * cs
        acc = jnp.zeros((cs, cout), _F32)
        for t, wt in enumerate(taps):
            dy, dx = divmod(t, 3)
            off = (dy - 1) * wb + (dx - 1)
            xs = _wrap_slice(x2, r0 + off, cs)
            acc = acc + jax.lax.dot(xs, wt, preferred_element_type=_F32)
        outs.append(acc)
    return jnp.concatenate(outs, axis=0) if nchunk > 1 else outs[0]


def _conv_b(x4, taps, scratch, stride=1):
    b, hb, wb, cin = x4.shape
    cout = taps[0].shape[1]
    ho, wo = hb // stride, wb // stride
    x2 = x4.reshape(b * hb * wb, cin)
    nj = max(1, cout // 128)
    cw = cout // nj
    accs = [jnp.zeros((b, ho, wo, cw), _F32) for _ in range(nj)]
    for t, wt in enumerate(taps):
        dy, dx = divmod(t, 3)
        y = jax.lax.dot(x2, wt, preferred_element_type=_F32)
        y4 = y.reshape(b, hb, wb, cout)
        for j in range(nj):
            scratch[:, 1:hb + 1, 1:wb + 1, 0:cw] = jax.lax.slice(
                y4, (0, 0, 0, j * cw), (b, hb, wb, (j + 1) * cw))
            if stride == 1:
                sl = scratch[:, pl.ds(dy, hb), pl.ds(dx, wb), pl.ds(0, cw)]
            else:
                sl = scratch[:, pl.ds(dy, ho, 2), pl.ds(dx, wo, 2),
                             pl.ds(0, cw)]
            accs[j] = accs[j] + sl
    return jnp.concatenate(accs, axis=3) if nj > 1 else accs[0]


def _expand4(v4, c):
    n = v4.shape[0]
    parts = [jnp.broadcast_to(jax.lax.slice(v4, (0, s), (n, s + 1)), (n, c))
             for s in range(4)]
    return jnp.concatenate(parts, axis=1)


def _sigmoid(z):
    return 1.0 / (1.0 + jnp.exp(-z))


def _mwhere(m, v):
    return jnp.where(m > 0, v, 0.0)


def _stem_body(din_ref, *rest):
    out_ref = rest[-1]
    it = iter(rest[:-1])
    nxt = lambda: next(it)[...]
    s = _HP * _HP
    taps = [nxt() for _ in range(9)]
    g, b = nxt(), nxt()
    din = din_ref[...]
    x2 = jax.lax.slice(din, (0, 0), (s, 4 * _CIN))
    x2 = _roll_conv(x2, taps, _HP)
    m = _expand4(jax.lax.slice(din, (0, 4 * _CIN), (s, 4 * _CIN + 4)), _BASE)
    x2 = x2 * m
    x2 = jnp.maximum(x2 * g + b, 0.0) * m
    out_ref[...] = x2


def _pair_body(x_ref, m_ref, *rest):
    out_ref = rest[-1]
    it = iter(rest[:-1])
    nxt = lambda: next(it)[...]
    g1, b1 = nxt(), nxt()
    t1 = [nxt() for _ in range(9)]
    g2, b2 = nxt(), nxt()
    t2 = [nxt() for _ in range(9)]
    wb = int(round(x_ref.shape[0] ** 0.5))
    a = _mwhere(m_ref[...], jnp.maximum(x_ref[...] * g1 + b1, 0.0))
    a = _mwhere(m_ref[...], _roll_conv(a, t1, wb))
    a = _mwhere(m_ref[...], jnp.maximum(a * g2 + b2, 0.0))
    a = _mwhere(m_ref[...], _roll_conv(a, t2, wb))
    out_ref[...] = a


def _seres_body(c_ref, x_ref, m_ref, *rest):
    out_ref = rest[-1]
    it = iter(rest[:-1])
    nxt = lambda: next(it)[...]
    fc1w, fc1b, fc2w, fc2b = nxt(), nxt(), nxt(), nxt()
    has_skip = (x_ref.shape[1] != c_ref.shape[1])
    skip = nxt() if has_skip else None
    inv = 1.0 / jnp.maximum(
        jnp.sum(m_ref[...].astype(_F32), axis=0, keepdims=True), 1.0)
    c = c_ref[...]
    pooled = jnp.sum(c, axis=0, keepdims=True) * inv
    h = jnp.maximum(jax.lax.dot(pooled, fc1w, preferred_element_type=_F32) + fc1b, 0.0)
    z = jax.lax.dot(h, fc2w, preferred_element_type=_F32) + fc2b
    out = c * _sigmoid(z)
    if skip is None:
        out = out + x_ref[...]
    else:
        out = out + _mwhere(m_ref[...], jax.lax.dot(
            x_ref[...], skip, preferred_element_type=_F32))
    out_ref[...] = out


def _block_p_body(x_ref, m_ref, *rest):
    out_ref = rest[-1]
    it = iter(rest[:-1])
    nxt = lambda: next(it)[...]
    g1, b1 = nxt(), nxt()
    t1 = [nxt() for _ in range(9)]
    g2, b2 = nxt(), nxt()
    t2 = [nxt() for _ in range(9)]
    fc1w, fc1b, fc2w, fc2b = nxt(), nxt(), nxt(), nxt()
    cin4, cout4 = x_ref.shape[1], t1[0].shape[1]
    cin4 = min(cin4, g1.shape[1])
    skip = nxt() if cin4 != cout4 else None
    wb = int(round(x_ref.shape[0] ** 0.5))
    s = x_ref.shape[0]
    m_out = m_ref[...]
    x2 = jax.lax.slice(x_ref[...], (0, 0), (s, cin4))
    if cin4 == cout4:
        m_in = m_out
    else:
        m_in = _expand4(jnp.concatenate(
            [jax.lax.slice(m_out, (0, smp * (cout4 // 4)),
                           (s, smp * (cout4 // 4) + 1)) for smp in range(4)],
            axis=1), cin4 // 4)
    a = _mwhere(m_in, jnp.maximum(x2 * g1 + b1, 0.0))
    a = _mwhere(m_out, _roll_conv(a, t1, wb))
    a = _mwhere(m_out, jnp.maximum(a * g2 + b2, 0.0))
    a = _mwhere(m_out, _roll_conv(a, t2, wb))
    inv = 1.0 / jnp.maximum(
        jnp.sum(m_out.astype(_F32), axis=0, keepdims=True), 1.0)
    pooled = jnp.sum(a, axis=0, keepdims=True) * inv
    h = jnp.maximum(jax.lax.dot(pooled, fc1w, preferred_element_type=_F32) + fc1b, 0.0)
    z = jax.lax.dot(h, fc2w, preferred_element_type=_F32) + fc2b
    out = a * _sigmoid(z)
    if skip is None:
        out = out + x2
    else:
        out = out + _mwhere(m_out, jax.lax.dot(
            x2, skip, preferred_element_type=_F32))
    out_ref[...] = out


def _down_p_body(x_ref, m_ref, *rest):
    out_ref, scratch = rest[-2], rest[-1]
    it = iter(rest[:-2])
    nxt = lambda: next(it)[...]
    taps = [nxt() for _ in range(9)]
    g, b = nxt(), nxt()
    c4 = taps[0].shape[1]
    c = c4 // 4
    s = x_ref.shape[0]
    wb = int(round(s ** 0.5))
    hb = wb
    ho, wo = hb // 2, wb // 2
    so = ho * wo
    scratch[...] = jnp.zeros(scratch.shape, _F32)
    m = m_ref[...]
    m4 = jnp.concatenate(
        [jax.lax.slice(m, (0, smp * c), (s, smp * c + 1)) for smp in range(4)],
        axis=1).astype(_F32)
    scratch[1:hb + 1, 1:wb + 1, 0:4] = m4.reshape(hb, wb, 4)
    pa = jnp.zeros((ho, wo, 4), _F32)
    for dy in range(3):
        for dx in range(3):
            pa = jnp.maximum(pa, scratch[pl.ds(dy, ho, 2), pl.ds(dx, wo, 2),
                                         pl.ds(0, 4)])
    m4o = (pa > 0).astype(_F32).reshape(so, 4)
    x2 = x_ref[...]
    nj = max(1, c4 // 128)
    cw = c4 // nj
    accs = [jnp.zeros((so, cw), _F32) for _ in range(nj)]
    for t, wt in enumerate(taps):
        dy, dx = divmod(t, 3)
        y = jax.lax.dot(x2, wt, preferred_element_type=_F32)
        y3 = y.reshape(hb, wb, c4)
        for j in range(nj):
            scratch[1:hb + 1, 1:wb + 1, 0:cw] = jax.lax.slice(
                y3, (0, 0, j * cw), (hb, wb, (j + 1) * cw))
            accs[j] = accs[j] + scratch[pl.ds(dy, ho, 2), pl.ds(dx, wo, 2),
                                        pl.ds(0, cw)].reshape(so, cw)
    acc = jnp.concatenate(accs, axis=1) if nj > 1 else accs[0]
    y = jnp.maximum(acc * g + b, 0.0) * _expand4(m4o, c)
    out_ref[...] = jnp.concatenate([y, m4o], axis=1)


def _seg_c_body(xb_ref, *rest):
    out_x, out_m, scratch = rest[-3], rest[-2], rest[-1]
    it = iter(rest[:-3])
    nxt = lambda: next(it)[...]
    hb = wb = 32
    c2 = _BASE * 2
    so = hb * wb
    xb = xb_ref[...]
    x4 = jnp.stack([jax.lax.slice(xb, (0, smp * c2), (so, (smp + 1) * c2))
                    .reshape(hb, wb, c2) for smp in range(4)])
    m4d = jnp.stack([jax.lax.slice(xb, (0, 4 * c2 + smp), (so, 4 * c2 + smp + 1))
                     .reshape(hb, wb, 1) for smp in range(4)])
    scratch[...] = jnp.zeros(scratch.shape, _F32)
    inv = 1.0 / jnp.maximum(jnp.sum(m4d, axis=(1, 2)), 1.0)
    x4 = _block_b(nxt, x4, m4d, inv, scratch)
    x4 = _block_b(nxt, x4, m4d, inv, scratch)
    x4 = _block_b(nxt, x4, m4d, inv, scratch)
    wd = [nxt() for _ in range(9)]
    gd, bd_ = nxt(), nxt()
    ho, wo = hb // 2, wb // 2
    scratch[:, 1:hb + 1, 1:wb + 1, 0:1] = m4d
    pa = jnp.zeros((4, ho, wo, 1), _F32)
    for dy in range(3):
        for dx in range(3):
            pa = jnp.maximum(pa, scratch[:, pl.ds(dy, ho, 2), pl.ds(dx, wo, 2),
                                         pl.ds(0, 1)])
    m4o = (pa > 0).astype(_F32)
    y = _conv_b(x4, wd, scratch, stride=2)
    y = jnp.maximum(y * gd + bd_, 0.0) * m4o
    out_x[...] = y
    out_m[...] = m4o


def _block_b(nxt, x4, m4d, inv, scratch):
    b, hb, wb, cin = x4.shape
    g1, b1 = nxt(), nxt()
    w1 = [nxt() for _ in range(9)]
    cout = w1[0].shape[1]
    g2, b2 = nxt(), nxt()
    w2 = [nxt() for _ in range(9)]
    fc1w, fc1b, fc2w, fc2b = nxt(), nxt(), nxt(), nxt()
    skip = nxt() if cin != cout else None
    out = jnp.maximum(x4 * g1 + b1, 0.0) * m4d
    out = _conv_b(out, w1, scratch) * m4d
    out = jnp.maximum(out * g2 + b2, 0.0) * m4d
    out = _conv_b(out, w2, scratch) * m4d
    pooled = jnp.sum(out, axis=(1, 2)) * inv
    h = jnp.maximum(jax.lax.dot(pooled, fc1w, preferred_element_type=_F32) + fc1b, 0.0)
    z = jax.lax.dot(h, fc2w, preferred_element_type=_F32) + fc2b
    out = out * _sigmoid(z)[:, None, None, :]
    if skip is None:
        ident = x4
    else:
        s2 = jax.lax.dot(x4.reshape(b * hb * wb, cin), skip,
                         preferred_element_type=_F32)
        ident = s2.reshape(b, hb, wb, cout) * m4d
    return out + ident


def _seg_d_body(x_ref, m_ref, *rest):
    out_x, scratch = rest[-2], rest[-1]
    it = iter(rest[:-2])
    nxt = lambda: next(it)[...]
    x4 = x_ref[...]
    m4d = m_ref[...]
    scratch[...] = jnp.zeros(scratch.shape, _F32)
    inv = 1.0 / jnp.maximum(jnp.sum(m4d, axis=(1, 2)), 1.0)
    x4 = _block_b(nxt, x4, m4d, inv, scratch)
    x4 = _block_b(nxt, x4, m4d, inv, scratch)
    x4 = _block_b(nxt, x4, m4d, inv, scratch)
    out_x[...] = x4


def _call(body, ins, out_shapes, out_dtypes=None, scratch_shape=None):
    if out_dtypes is None:
        out_dtypes = [_F32] * len(out_shapes)
    outs = [jax.ShapeDtypeStruct(s, d) for s, d in zip(out_shapes, out_dtypes)]
    return pl.pallas_call(
        body,
        out_shape=outs if len(outs) > 1 else outs[0],
        scratch_shapes=([pltpu.VMEM(scratch_shape, _F32)]
                        if scratch_shape is not None else []),
        interpret=_INTERPRET,
    )(*ins)


def kernel(features, indices, params):
    c1, c2, c3, c4 = _BASE, _BASE * 2, _BASE * 4, _BASE * 8
    b, yy, xx = indices[:, 0], indices[:, 1], indices[:, 2]
    dense = jnp.zeros((_B, _HP, _HP, _CIN), _F32).at[b, yy, xx].set(features)
    mask = jnp.zeros((_B, _HP, _HP, 1), _F32).at[b, yy, xx].set(1.0)
    s0 = _HP * _HP
    densep = dense.transpose(1, 2, 0, 3).reshape(s0, _B * _CIN)
    maskp = mask.transpose(1, 2, 0, 3).reshape(s0, _B)
    din = jnp.concatenate([densep, maskp], axis=1)
    mex1 = jnp.repeat(maskp, c1, axis=1).astype(_BF16)

    pst = _taps_bd(params['stem']['w']) + [_t4(params['stem']['bn']['g']),
                                           _t4(params['stem']['bn']['b'])]
    x = _call(_stem_body, [din] + pst, [(s0, 4 * c1)])
    for bp in params['stage1']:
        ppair = ([_t4(bp['bn1']['g']), _t4(bp['bn1']['b'])] + _taps_bd(bp['w1'])
                 + [_t4(bp['bn2']['g']), _t4(bp['bn2']['b'])] + _taps_bd(bp['w2']))
        cpair = _call(_pair_body, [x, mex1] + ppair, [(s0, 4 * c1)])
        pse = [_bd(bp['fc1_w']), _t4(bp['fc1_b']), _bd(bp['fc2_w']),
               _t4(bp['fc2_b'])]
        x = _call(_seres_body, [cpair, x, mex1] + pse, [(s0, 4 * c1)])
    pdn = _taps_bd(params['down1']['w']) + [_t4(params['down1']['bn']['g']),
                                            _t4(params['down1']['bn']['b'])]
    xa = _call(_down_p_body, [x, mex1] + pdn, [(64 * 64, 4 * c1 + 4)],
               scratch_shape=(_HP + 2, _HP + 2, 128))

    s1 = 64 * 64
    x = xa[:, :4 * c1]
    mex2 = jnp.repeat(xa[:, 4 * c1:], c2, axis=1).astype(_BF16)
    for bp, (ci, co) in zip(params['stage2'], [(c1, c2), (c2, c2), (c2, c2)]):
        pblk = ([_t4(bp['bn1']['g']), _t4(bp['bn1']['b'])] + _taps_bd(bp['w1'])
                + [_t4(bp['bn2']['g']), _t4(bp['bn2']['b'])] + _taps_bd(bp['w2'])
                + [_bd(bp['fc1_w']), _t4(bp['fc1_b']), _bd(bp['fc2_w']),
                   _t4(bp['fc2_b'])])
        if ci != co:
            pblk.append(_bd(bp['skip_w'].reshape(ci, co)))
        x = _call(_block_p_body, [x, mex2] + pblk, [(s1, 4 * co)])
    pdn2 = _taps_bd(params['down2']['w']) + [_t4(params['down2']['bn']['g']),
                                             _t4(params['down2']['bn']['b'])]
    xb = _call(_down_p_body, [x, mex2] + pdn2, [(32 * 32, 4 * c2 + 4)],
               scratch_shape=(66, 66, 128))

    pc = []
    for bp, (ci, co) in zip(params['stage3'], [(c2, c3), (c3, c3), (c3, c3)]):
        pc += [bp['bn1']['g'].reshape(1, -1), bp['bn1']['b'].reshape(1, -1)]
        pc += _taps(bp['w1'])
        pc += [bp['bn2']['g'].reshape(1, -1), bp['bn2']['b'].reshape(1, -1)]
        pc += _taps(bp['w2'])
        pc += [bp['fc1_w'], bp['fc1_b'].reshape(1, -1),
               bp['fc2_w'], bp['fc2_b'].reshape(1, -1)]
        if ci != co:
            pc.append(bp['skip_w'].reshape(ci, co))
    pc += _taps(params['down3']['w'])
    pc += [params['down3']['bn']['g'].reshape(1, -1),
           params['down3']['bn']['b'].reshape(1, -1)]
    xc, mc = _call(_seg_c_body, [xb] + pc,
                   [(_B, 16, 16, c3), (_B, 16, 16, 1)],
                   scratch_shape=(_B, 34, 34, 128))

    pd = []
    for bp, (ci, co) in zip(params['stage4'], [(c3, c4), (c4, c4), (c4, c4)]):
        pd += [bp['bn1']['g'].reshape(1, -1), bp['bn1']['b'].reshape(1, -1)]
        pd += _taps(bp['w1'])
        pd += [bp['bn2']['g'].reshape(1, -1), bp['bn2']['b'].reshape(1, -1)]
        pd += _taps(bp['w2'])
        pd += [bp['fc1_w'], bp['fc1_b'].reshape(1, -1),
               bp['fc2_w'], bp['fc2_b'].reshape(1, -1)]
        if ci != co:
            pd.append(bp['skip_w'].reshape(ci, co))
    out = _call(_seg_d_body, [xc, mc] + pd, [(_B, 16, 16, c4)],
                scratch_shape=(_B, 18, 18, 128))
    return out
